# Initial kernel scaffold; baseline (speedup 1.0000x reference)
import jax
import jax.numpy as jnp
from jax.experimental import pallas as pl


def kernel(x, edge_index, batch, W1, att_src1, att_dst1, bias1, gamma1, beta1, W2, att_src2, att_dst2, bias2, gamma2, beta2, W3, att_src3, att_dst3, bias3, gamma3, beta3):
    raise NotImplementedError("write your pallas kernel here")


# speedup vs baseline: 1.0000x; 1.0000x over previous
import jax
import jax.numpy as jnp
from jax.experimental import pallas as pl

N = 10000
NG = 64


def _gat_conv(x, src, dst, W, att_src, att_dst, bias):
    n = x.shape[0]
    h = x @ W
    a_src = (h * att_src).sum(-1)
    a_dst = (h * att_dst).sum(-1)
    alpha = jax.nn.leaky_relu(a_src[src] + a_dst[dst], 0.2)
    m = jax.lax.stop_gradient(jax.ops.segment_max(alpha, dst, num_segments=n))
    ex = jnp.exp(alpha - m[dst])
    denom = jax.ops.segment_sum(ex, dst, num_segments=n)
    coef = ex / (denom[dst] + 1e-16)
    out = jax.ops.segment_sum(coef[:, None] * h[src], dst, num_segments=n)
    return out + bias


def _bn(x, gamma, beta):
    mu = x.mean(0)
    var = x.var(0)
    return gamma * (x - mu) * jax.lax.rsqrt(var + 1e-5) + beta


def kernel(x, edge_index, batch, W1, att_src1, att_dst1, bias1, gamma1, beta1, W2, att_src2, att_dst2, bias2, gamma2, beta2, W3, att_src3, att_dst3, bias3, gamma3, beta3):
    params = [(W1, att_src1, att_dst1, bias1, gamma1, beta1),
              (W2, att_src2, att_dst2, bias2, gamma2, beta2),
              (W3, att_src3, att_dst3, bias3, gamma3, beta3)]
    n = x.shape[0]
    loop = jnp.arange(n, dtype=edge_index.dtype)
    src = jnp.concatenate([edge_index[0], loop])
    dst = jnp.concatenate([edge_index[1], loop])
    xs = []
    h = x
    for (W, a_s, a_d, b, g, be) in params:
        h = _gat_conv(h, src, dst, W, a_s, a_d, b)
        h = jax.nn.elu(h)
        h = _bn(h, g, be)
        xs.append(h)
    pooled = [jax.ops.segment_sum(t, batch, num_segments=NG) for t in xs]

    def _id(x_ref, o_ref):
        o_ref[...] = x_ref[...]

    h = pl.pallas_call(_id, out_shape=jax.ShapeDtypeStruct(h.shape, h.dtype))(h)
    return (jnp.concatenate(pooled, axis=1), h)


# speedup vs baseline: 22.2726x; 22.2725x over previous
import functools

import jax
import jax.numpy as jnp
from jax import lax
from jax.experimental import pallas as pl
from jax.experimental.pallas import tpu as pltpu
from jax.experimental.pallas import tpu_sc as plsc

N = 10000
E = 320000
D = 128
NG = 64
NP = 10240
CHUNK = 128
NTILES = 32
NCHUNK = 81
EP = NTILES * NCHUNK * CHUNK
STRIPE = NP // 16
PAD_NEG = -1e9
PAD_POS = 1e9


def _pre_body(h_ref, w_ref, asv_ref, adv_ref, hw_out, as_out, ad_out, m_out):
    hw = jnp.dot(h_ref[...], w_ref[...], preferred_element_type=jnp.float32)
    hw_out[...] = hw
    a_s = jnp.sum(hw * asv_ref[...], axis=1, keepdims=True)
    a_d = jnp.sum(hw * adv_ref[...], axis=1, keepdims=True)
    valid = lax.broadcasted_iota(jnp.int32, (NP, 1), 0) < N
    as_out[...] = a_s
    ad_out[...] = jnp.where(valid, a_d, PAD_NEG)
    max_as = jnp.max(jnp.where(valid, a_s, PAD_NEG))
    t = max_as + a_d
    m_out[...] = jnp.where(valid, jnp.maximum(t, 0.2 * t), PAD_POS)


_pre = pl.pallas_call(
    _pre_body,
    out_shape=(
        jax.ShapeDtypeStruct((NP, D), jnp.float32),
        jax.ShapeDtypeStruct((NP, 1), jnp.float32),
        jax.ShapeDtypeStruct((NP, 1), jnp.float32),
        jax.ShapeDtypeStruct((NP, 1), jnp.float32),
    ),
)


def _edge_body(h_hbm, src_hbm, dst_hbm, asrc_hbm, adst_hbm, m_hbm,
               z2_hbm, z1_hbm, out_hbm, den_hbm,
               asrc_v, adst_v, m_v, sidx_v, didx_v, rows_v, ex_v,
               acc_sh, den_sh, sem):
    cid = lax.axis_index("c")
    sid = lax.axis_index("s")
    wid = cid * 16 + sid
    rbase = sid * STRIPE

    pltpu.sync_copy(asrc_hbm, asrc_v)
    pltpu.sync_copy(adst_hbm, adst_v)
    pltpu.sync_copy(m_hbm, m_v)
    pltpu.sync_copy(z2_hbm, acc_sh.at[pl.ds(rbase, STRIPE)])
    pltpu.sync_copy(z1_hbm, den_sh.at[pl.ds(rbase, STRIPE)])
    plsc.subcore_barrier()

    def chunk_body(c, carry):
        base = (wid * NCHUNK + c) * CHUNK
        pltpu.sync_copy(src_hbm.at[pl.ds(base, CHUNK)], sidx_v)
        pltpu.sync_copy(dst_hbm.at[pl.ds(base, CHUNK)], didx_v)
        cp = pltpu.async_copy(h_hbm.at[sidx_v], rows_v, sem)
        for g in range(CHUNK // 16):
            sv = sidx_v[pl.ds(g * 16, 16)]
            dv = didx_v[pl.ds(g * 16, 16)]
            asv = plsc.load_gather(asrc_v, [sv])
            adv = plsc.load_gather(adst_v, [dv])
            mv = plsc.load_gather(m_v, [dv])
            t = asv + adv
            ex_v[pl.ds(g * 16, 16)] = jnp.exp(jnp.maximum(t, 0.2 * t) - mv)
        cp.wait()

        def scale_body(e, carry2):
            bex = plsc.load_gather(ex_v, [jnp.full((16,), 0, jnp.int32) + e])
            for j in range(D // 16):
                rows_v[e, pl.ds(j * 16, 16)] = rows_v[e, pl.ds(j * 16, 16)] * bex
            return carry2

        lax.fori_loop(0, CHUNK, scale_body, 0)
        pltpu.sync_copy(rows_v, acc_sh.at[didx_v], add=True)
        pltpu.sync_copy(ex_v, den_sh.at[didx_v], add=True)
        return carry

    lax.fori_loop(0, NCHUNK, chunk_body, 0)
    plsc.subcore_barrier()
    pltpu.sync_copy(acc_sh.at[pl.ds(rbase, STRIPE)],
                    out_hbm.at[cid, pl.ds(rbase, STRIPE)])
    pltpu.sync_copy(den_sh.at[pl.ds(rbase, STRIPE)],
                    den_hbm.at[cid, pl.ds(rbase, STRIPE)])


_edge = functools.partial(
    pl.kernel,
    out_type=(
        jax.ShapeDtypeStruct((2, NP, D), jnp.float32),
        jax.ShapeDtypeStruct((2, NP), jnp.float32),
    ),
    mesh=plsc.VectorSubcoreMesh(core_axis_name="c", subcore_axis_name="s"),
    compiler_params=pltpu.CompilerParams(needs_layout_passes=False),
    scratch_types=[
        pltpu.VMEM((NP,), jnp.float32),
        pltpu.VMEM((NP,), jnp.float32),
        pltpu.VMEM((NP,), jnp.float32),
        pltpu.VMEM((CHUNK,), jnp.int32),
        pltpu.VMEM((CHUNK,), jnp.int32),
        pltpu.VMEM((CHUNK, D), jnp.float32),
        pltpu.VMEM((CHUNK,), jnp.float32),
        pltpu.VMEM_SHARED((NP, D), jnp.float32),
        pltpu.VMEM_SHARED((NP,), jnp.float32),
        pltpu.SemaphoreType.DMA,
    ],
)(_edge_body)


def _post_body(a0_ref, a1_ref, d0_ref, d1_ref, bias_ref, gamma_ref,
               beta_ref, batch_ref, h_out, pool_out):
    acc = a0_ref[...] + a1_ref[...]
    den = d0_ref[...] + d1_ref[...]
    y = acc / (den + 1e-16) + bias_ref[...]
    y = jnp.where(y > 0, y, jnp.exp(jnp.minimum(y, 0.0)) - 1.0)
    valid = lax.broadcasted_iota(jnp.int32, (NP, D), 0) < N
    y = jnp.where(valid, y, 0.0)
    mu = jnp.sum(y, axis=0, keepdims=True) / N
    var = jnp.sum(y * y, axis=0, keepdims=True) / N - mu * mu
    hn = gamma_ref[...] * (y - mu) * lax.rsqrt(var + 1e-5) + beta_ref[...]
    hn = jnp.where(valid, hn, 0.0)
    h_out[...] = hn
    onehot = (lax.broadcasted_iota(jnp.int32, (NG, NP), 0)
              == batch_ref[...]).astype(jnp.float32)
    pool_out[...] = jnp.dot(onehot, hn, preferred_element_type=jnp.float32)


_post = pl.pallas_call(
    _post_body,
    out_shape=(
        jax.ShapeDtypeStruct((NP, D), jnp.float32),
        jax.ShapeDtypeStruct((NG, D), jnp.float32),
    ),
)


def kernel(x, edge_index, batch, W1, att_src1, att_dst1, bias1, gamma1, beta1,
           W2, att_src2, att_dst2, bias2, gamma2, beta2,
           W3, att_src3, att_dst3, bias3, gamma3, beta3):
    params = [(W1, att_src1, att_dst1, bias1, gamma1, beta1),
              (W2, att_src2, att_dst2, bias2, gamma2, beta2),
              (W3, att_src3, att_dst3, bias3, gamma3, beta3)]
    loop = jnp.arange(N, dtype=jnp.int32)
    srcp = jnp.concatenate(
        [edge_index[0], loop, jnp.zeros((EP - E - N,), jnp.int32)])
    dstp = jnp.concatenate(
        [edge_index[1], loop, jnp.full((EP - E - N,), N, jnp.int32)])
    batch2 = jnp.pad(batch, (0, NP - N), constant_values=NG).reshape(1, NP)
    z2 = jnp.zeros((STRIPE, D), jnp.float32)
    z1 = jnp.zeros((STRIPE,), jnp.float32)

    h = jnp.pad(x, ((0, NP - N), (0, 0)))
    pooled = []
    for (W, a_s, a_d, b, g, be) in params:
        hw, asrc_t, adst_t, m_t = _pre(h, W, a_s.reshape(1, D),
                                       a_d.reshape(1, D))
        acc, den = _edge(hw, srcp, dstp, asrc_t.reshape(NP),
                         adst_t.reshape(NP), m_t.reshape(NP), z2, z1)
        h, pool_l = _post(acc[0], acc[1],
                          den[0].reshape(NP, 1), den[1].reshape(NP, 1),
                          b.reshape(1, D), g.reshape(1, D), be.reshape(1, D),
                          batch2)
        pooled.append(pool_l)
    return jnp.concatenate(pooled, axis=1), h[:N]


# speedup vs baseline: 26.3087x; 1.1812x over previous
import functools

import jax
import jax.numpy as jnp
from jax import lax
from jax.experimental import pallas as pl
from jax.experimental.pallas import tpu as pltpu
from jax.experimental.pallas import tpu_sc as plsc

N = 10000
E = 320000
D = 128
NG = 64
NP = 10240
CHUNK = 64
NTILES = 32
NCHUNK = 162
EP = NTILES * NCHUNK * CHUNK
STRIPE = NP // 16
PAD_NEG = -1e9
PAD_POS = 1e9


def _pre_body(h_ref, w_ref, asv_ref, adv_ref, hw_out, as_out, ad_out, mx_out):
    hw = jnp.dot(h_ref[...], w_ref[...], preferred_element_type=jnp.float32)
    hw_out[...] = hw
    a_s = jnp.sum(hw * asv_ref[...], axis=1, keepdims=True)
    a_d = jnp.sum(hw * adv_ref[...], axis=1, keepdims=True)
    valid = lax.broadcasted_iota(jnp.int32, (NP, 1), 0) < N
    as_out[...] = jnp.where(valid, a_s, PAD_NEG)
    ad_out[...] = jnp.where(valid, a_d, PAD_NEG)
    max_as = jnp.max(jnp.where(valid, a_s, PAD_NEG))
    mx_out[...] = jnp.zeros((1, D), jnp.float32) + max_as


_pre = pl.pallas_call(
    _pre_body,
    out_shape=(
        jax.ShapeDtypeStruct((NP, D), jnp.float32),
        jax.ShapeDtypeStruct((NP, 1), jnp.float32),
        jax.ShapeDtypeStruct((NP, 1), jnp.float32),
        jax.ShapeDtypeStruct((1, D), jnp.float32),
    ),
)


NBUF = 3


def _edge_body(h_hbm, src_hbm, dst_hbm, asrc_hbm, adst_hbm, mx_hbm,
               z2_hbm, z1_hbm, out_hbm, den_hbm,
               asrc_v, adst_v, mx_v, sidx_v, didx_v, rows_v, ex_v,
               acc_sh, den_sh, gsem, ssem):
    cid = lax.axis_index("c")
    sid = lax.axis_index("s")
    wid = cid * 16 + sid
    rbase = sid * STRIPE
    ebase = wid * NCHUNK * CHUNK

    pltpu.sync_copy(asrc_hbm, asrc_v)
    pltpu.sync_copy(adst_hbm, adst_v)
    pltpu.sync_copy(mx_hbm, mx_v)
    pltpu.sync_copy(z2_hbm, acc_sh.at[pl.ds(rbase, STRIPE)])
    pltpu.sync_copy(z1_hbm, den_sh.at[pl.ds(rbase, STRIPE)])
    plsc.subcore_barrier()

    def load_idx(c, b):
        pltpu.sync_copy(src_hbm.at[pl.ds(ebase + c * CHUNK, CHUNK)],
                        sidx_v.at[b])
        pltpu.sync_copy(dst_hbm.at[pl.ds(ebase + c * CHUNK, CHUNK)],
                        didx_v.at[b])

    def gather(b):
        pltpu.async_copy(h_hbm.at[sidx_v.at[b]], rows_v.at[b], gsem.at[b])

    def wait_scatters(b):
        pltpu.make_async_copy(rows_v.at[b], acc_sh.at[didx_v.at[b]],
                              ssem.at[b]).wait()
        pltpu.make_async_copy(ex_v.at[b], den_sh.at[didx_v.at[b]],
                              ssem.at[b]).wait()

    load_idx(0, 0)
    gather(0)
    mx = mx_v[pl.ds(0, 16)]

    def outer_body(o, carry):
        for b in range(NBUF):
            c = o * NBUF + b
            bn = (b + 1) % NBUF

            for g in range(CHUNK // 16):
                sv = sidx_v[b, pl.ds(g * 16, 16)]
                dv = didx_v[b, pl.ds(g * 16, 16)]
                asv = plsc.load_gather(asrc_v, [sv])
                adv = plsc.load_gather(adst_v, [dv])
                tb = mx + adv
                mv = jnp.maximum(tb, 0.2 * tb)
                t = asv + adv
                ex_v[b, pl.ds(g * 16, 16)] = (
                    jnp.exp(jnp.maximum(t, 0.2 * t) - mv))

            @pl.when(c + 1 < NCHUNK)
            def _():
                load_idx(c + 1, bn)
                gather(bn)

            pltpu.make_async_copy(h_hbm.at[sidx_v.at[b]], rows_v.at[b],
                                  gsem.at[b]).wait()

            def scale_body(e4, carry2):
                for u in range(4):
                    e = e4 * 4 + u
                    bex = plsc.load_gather(
                        ex_v.at[b], [jnp.full((16,), 0, jnp.int32) + e])
                    for j in range(D // 16):
                        rows_v[b, e, pl.ds(j * 16, 16)] = (
                            rows_v[b, e, pl.ds(j * 16, 16)] * bex)
                return carry2

            lax.fori_loop(0, CHUNK // 4, scale_body, 0)
            pltpu.sync_copy(rows_v.at[b], acc_sh.at[didx_v.at[b]], add=True)
            pltpu.sync_copy(ex_v.at[b], den_sh.at[didx_v.at[b]], add=True)
        return carry

    lax.fori_loop(0, NCHUNK // NBUF, outer_body, 0)
    plsc.subcore_barrier()
    pltpu.sync_copy(acc_sh.at[pl.ds(rbase, STRIPE)],
                    out_hbm.at[cid, pl.ds(rbase, STRIPE)])
    pltpu.sync_copy(den_sh.at[pl.ds(rbase, STRIPE)],
                    den_hbm.at[cid, pl.ds(rbase, STRIPE)])


_edge = functools.partial(
    pl.kernel,
    out_type=(
        jax.ShapeDtypeStruct((2, NP, D), jnp.float32),
        jax.ShapeDtypeStruct((2, NP), jnp.float32),
    ),
    mesh=plsc.VectorSubcoreMesh(core_axis_name="c", subcore_axis_name="s"),
    compiler_params=pltpu.CompilerParams(needs_layout_passes=False),
    scratch_types=[
        pltpu.VMEM((NP,), jnp.float32),
        pltpu.VMEM((NP,), jnp.float32),
        pltpu.VMEM((16,), jnp.float32),
        pltpu.VMEM((NBUF, CHUNK), jnp.int32),
        pltpu.VMEM((NBUF, CHUNK), jnp.int32),
        pltpu.VMEM((NBUF, CHUNK, D), jnp.float32),
        pltpu.VMEM((NBUF, CHUNK), jnp.float32),
        pltpu.VMEM_SHARED((NP, D), jnp.float32),
        pltpu.VMEM_SHARED((NP,), jnp.float32),
        pltpu.SemaphoreType.DMA((NBUF,)),
        pltpu.SemaphoreType.DMA((NBUF,)),
    ],
)(_edge_body)


def _post_body(a0_ref, a1_ref, d0_ref, d1_ref, bias_ref, gamma_ref,
               beta_ref, batch_ref, h_out, pool_out):
    acc = a0_ref[...] + a1_ref[...]
    den = d0_ref[...] + d1_ref[...]
    y = acc / (den + 1e-16) + bias_ref[...]
    y = jnp.where(y > 0, y, jnp.exp(jnp.minimum(y, 0.0)) - 1.0)
    valid = lax.broadcasted_iota(jnp.int32, (NP, D), 0) < N
    y = jnp.where(valid, y, 0.0)
    mu = jnp.sum(y, axis=0, keepdims=True) / N
    var = jnp.sum(y * y, axis=0, keepdims=True) / N - mu * mu
    hn = gamma_ref[...] * (y - mu) * lax.rsqrt(var + 1e-5) + beta_ref[...]
    hn = jnp.where(valid, hn, 0.0)
    h_out[...] = hn
    onehot = (lax.broadcasted_iota(jnp.int32, (NG, NP), 0)
              == batch_ref[...]).astype(jnp.float32)
    pool_out[...] = jnp.dot(onehot, hn, preferred_element_type=jnp.float32)


_post = pl.pallas_call(
    _post_body,
    out_shape=(
        jax.ShapeDtypeStruct((NP, D), jnp.float32),
        jax.ShapeDtypeStruct((NG, D), jnp.float32),
    ),
)


def kernel(x, edge_index, batch, W1, att_src1, att_dst1, bias1, gamma1, beta1,
           W2, att_src2, att_dst2, bias2, gamma2, beta2,
           W3, att_src3, att_dst3, bias3, gamma3, beta3):
    params = [(W1, att_src1, att_dst1, bias1, gamma1, beta1),
              (W2, att_src2, att_dst2, bias2, gamma2, beta2),
              (W3, att_src3, att_dst3, bias3, gamma3, beta3)]
    loop = jnp.arange(N, dtype=jnp.int32)
    srcp = jnp.concatenate(
        [edge_index[0], loop, jnp.full((EP - E - N,), N, jnp.int32)])
    dstp = jnp.concatenate(
        [edge_index[1], loop, jnp.full((EP - E - N,), N, jnp.int32)])
    batch2 = jnp.pad(batch, (0, NP - N), constant_values=NG).reshape(1, NP)
    z2 = jnp.zeros((STRIPE, D), jnp.float32)
    z1 = jnp.zeros((STRIPE,), jnp.float32)

    h = jnp.pad(x, ((0, NP - N), (0, 0)))
    pooled = []
    for (W, a_s, a_d, b, g, be) in params:
        hw, asrc_t, adst_t, mx_t = _pre(h, W, a_s.reshape(1, D),
                                        a_d.reshape(1, D))
        acc, den = _edge(hw, srcp, dstp, asrc_t.reshape(NP),
                         adst_t.reshape(NP), mx_t.reshape(D)[:16], z2, z1)
        h, pool_l = _post(acc[0], acc[1],
                          den[0].reshape(NP, 1), den[1].reshape(NP, 1),
                          b.reshape(1, D), g.reshape(1, D), be.reshape(1, D),
                          batch2)
        pooled.append(pool_l)
    return jnp.concatenate(pooled, axis=1), h[:N]


# speedup vs baseline: 29.3330x; 1.1150x over previous
import functools

import jax
import jax.numpy as jnp
from jax import lax
from jax.experimental import pallas as pl
from jax.experimental.pallas import tpu as pltpu
from jax.experimental.pallas import tpu_sc as plsc

N = 10000
E = 320000
D = 128
NG = 64
NP = 10240
CHUNK = 64
NTILES = 32
NCHUNK = 162
EP = NTILES * NCHUNK * CHUNK
STRIPE = NP // 16
PAD_NEG = -1e9
PAD_POS = 1e9


def _pre_body(h_ref, w_ref, asv_ref, adv_ref, hw_out, as_out, ad_out, mx_out):
    hw = jnp.dot(h_ref[...], w_ref[...], preferred_element_type=jnp.float32)
    hw_out[...] = hw
    a_s = jnp.sum(hw * asv_ref[...], axis=1, keepdims=True)
    a_d = jnp.sum(hw * adv_ref[...], axis=1, keepdims=True)
    valid = lax.broadcasted_iota(jnp.int32, (NP, 1), 0) < N
    as_out[...] = jnp.where(valid, a_s, PAD_NEG)
    ad_out[...] = jnp.where(valid, a_d, PAD_NEG)
    max_as = jnp.max(jnp.where(valid, a_s, PAD_NEG))
    mx_out[...] = jnp.zeros((1, D), jnp.float32) + max_as


_pre = pl.pallas_call(
    _pre_body,
    out_shape=(
        jax.ShapeDtypeStruct((NP, D), jnp.float32),
        jax.ShapeDtypeStruct((NP, 1), jnp.float32),
        jax.ShapeDtypeStruct((NP, 1), jnp.float32),
        jax.ShapeDtypeStruct((1, D), jnp.float32),
    ),
)


NBUF = 3


def _edge_body(h_hbm, src_hbm, dst_hbm, asrc_hbm, adst_hbm, mx_hbm,
               z2_hbm, z1_hbm, out_hbm, den_hbm,
               asrc_v, adst_v, mx_v, sidx_v, didx_v, rows_v, ex_v,
               acc_sh, den_sh, gsem, ssem):
    cid = lax.axis_index("c")
    sid = lax.axis_index("s")
    wid = cid * 16 + sid
    rbase = sid * STRIPE
    ebase = wid * NCHUNK * CHUNK

    pltpu.sync_copy(asrc_hbm, asrc_v)
    pltpu.sync_copy(adst_hbm, adst_v)
    pltpu.sync_copy(mx_hbm, mx_v)
    pltpu.sync_copy(z2_hbm, acc_sh.at[pl.ds(rbase, STRIPE)])
    pltpu.sync_copy(z1_hbm, den_sh.at[pl.ds(rbase, STRIPE)])
    plsc.subcore_barrier()

    def load_idx(c, b):
        pltpu.sync_copy(src_hbm.at[pl.ds(ebase + c * CHUNK, CHUNK)],
                        sidx_v.at[b])
        pltpu.sync_copy(dst_hbm.at[pl.ds(ebase + c * CHUNK, CHUNK)],
                        didx_v.at[b])

    def gather(b):
        pltpu.async_copy(h_hbm.at[sidx_v.at[b]], rows_v.at[b], gsem.at[b])

    def wait_scatter(b):
        pltpu.make_async_copy(rows_v.at[b], acc_sh.at[didx_v.at[b]],
                              ssem.at[b]).wait()

    load_idx(0, 0)
    gather(0)
    mx = mx_v[pl.ds(0, 16)]

    def outer_body(o, carry):
        for b in range(NBUF):
            c = o * NBUF + b
            bn = (b + 1) % NBUF

            for g in range(CHUNK // 16):
                sv = sidx_v[b, pl.ds(g * 16, 16)]
                dv = didx_v[b, pl.ds(g * 16, 16)]
                asv = plsc.load_gather(asrc_v, [sv])
                adv = plsc.load_gather(adst_v, [dv])
                tb = mx + adv
                mv = jnp.maximum(tb, 0.2 * tb)
                t = asv + adv
                ex_v[b, pl.ds(g * 16, 16)] = (
                    jnp.exp(jnp.maximum(t, 0.2 * t) - mv))

            @pl.when(jnp.logical_and(c + 1 < NCHUNK, c >= NBUF - 1))
            def _():
                wait_scatter(bn)

            @pl.when(c + 1 < NCHUNK)
            def _():
                load_idx(c + 1, bn)
                gather(bn)

            pltpu.make_async_copy(h_hbm.at[sidx_v.at[b]], rows_v.at[b],
                                  gsem.at[b]).wait()

            def scale_body(e4, carry2):
                for u in range(4):
                    e = e4 * 4 + u
                    bex = plsc.load_gather(
                        ex_v.at[b], [jnp.full((16,), 0, jnp.int32) + e])
                    for j in range(D // 16):
                        rows_v[b, e, pl.ds(j * 16, 16)] = (
                            rows_v[b, e, pl.ds(j * 16, 16)] * bex)
                return carry2

            lax.fori_loop(0, CHUNK // 4, scale_body, 0)
            pltpu.async_copy(rows_v.at[b], acc_sh.at[didx_v.at[b]],
                             ssem.at[b], add=True)
            pltpu.sync_copy(ex_v.at[b], den_sh.at[didx_v.at[b]], add=True)
        return carry

    lax.fori_loop(0, NCHUNK // NBUF, outer_body, 0)
    for k in range(NBUF - 1, 0, -1):
        wait_scatter((NCHUNK - k) % NBUF)
    plsc.subcore_barrier()
    pltpu.sync_copy(acc_sh.at[pl.ds(rbase, STRIPE)],
                    out_hbm.at[cid, pl.ds(rbase, STRIPE)])
    pltpu.sync_copy(den_sh.at[pl.ds(rbase, STRIPE)],
                    den_hbm.at[cid, pl.ds(rbase, STRIPE)])


_edge = functools.partial(
    pl.kernel,
    out_type=(
        jax.ShapeDtypeStruct((2, NP, D), jnp.float32),
        jax.ShapeDtypeStruct((2, NP), jnp.float32),
    ),
    mesh=plsc.VectorSubcoreMesh(core_axis_name="c", subcore_axis_name="s"),
    compiler_params=pltpu.CompilerParams(needs_layout_passes=False),
    scratch_types=[
        pltpu.VMEM((NP,), jnp.float32),
        pltpu.VMEM((NP,), jnp.float32),
        pltpu.VMEM((16,), jnp.float32),
        pltpu.VMEM((NBUF, CHUNK), jnp.int32),
        pltpu.VMEM((NBUF, CHUNK), jnp.int32),
        pltpu.VMEM((NBUF, CHUNK, D), jnp.float32),
        pltpu.VMEM((NBUF, CHUNK), jnp.float32),
        pltpu.VMEM_SHARED((NP, D), jnp.float32),
        pltpu.VMEM_SHARED((NP,), jnp.float32),
        pltpu.SemaphoreType.DMA((NBUF,)),
        pltpu.SemaphoreType.DMA((NBUF,)),
    ],
)(_edge_body)


def _post_body(a0_ref, a1_ref, d0_ref, d1_ref, bias_ref, gamma_ref,
               beta_ref, batch_ref, h_out, pool_out):
    acc = a0_ref[...] + a1_ref[...]
    den = d0_ref[...] + d1_ref[...]
    y = acc / (den + 1e-16) + bias_ref[...]
    y = jnp.where(y > 0, y, jnp.exp(jnp.minimum(y, 0.0)) - 1.0)
    valid = lax.broadcasted_iota(jnp.int32, (NP, D), 0) < N
    y = jnp.where(valid, y, 0.0)
    mu = jnp.sum(y, axis=0, keepdims=True) / N
    var = jnp.sum(y * y, axis=0, keepdims=True) / N - mu * mu
    hn = gamma_ref[...] * (y - mu) * lax.rsqrt(var + 1e-5) + beta_ref[...]
    hn = jnp.where(valid, hn, 0.0)
    h_out[...] = hn
    onehot = (lax.broadcasted_iota(jnp.int32, (NG, NP), 0)
              == batch_ref[...]).astype(jnp.float32)
    pool_out[...] = jnp.dot(onehot, hn, preferred_element_type=jnp.float32)


_post = pl.pallas_call(
    _post_body,
    out_shape=(
        jax.ShapeDtypeStruct((NP, D), jnp.float32),
        jax.ShapeDtypeStruct((NG, D), jnp.float32),
    ),
)


def kernel(x, edge_index, batch, W1, att_src1, att_dst1, bias1, gamma1, beta1,
           W2, att_src2, att_dst2, bias2, gamma2, beta2,
           W3, att_src3, att_dst3, bias3, gamma3, beta3):
    params = [(W1, att_src1, att_dst1, bias1, gamma1, beta1),
              (W2, att_src2, att_dst2, bias2, gamma2, beta2),
              (W3, att_src3, att_dst3, bias3, gamma3, beta3)]
    loop = jnp.arange(N, dtype=jnp.int32)
    srcp = jnp.concatenate(
        [edge_index[0], loop, jnp.full((EP - E - N,), N, jnp.int32)])
    dstp = jnp.concatenate(
        [edge_index[1], loop, jnp.full((EP - E - N,), N, jnp.int32)])
    batch2 = jnp.pad(batch, (0, NP - N), constant_values=NG).reshape(1, NP)
    z2 = jnp.zeros((STRIPE, D), jnp.float32)
    z1 = jnp.zeros((STRIPE,), jnp.float32)

    h = jnp.pad(x, ((0, NP - N), (0, 0)))
    pooled = []
    for (W, a_s, a_d, b, g, be) in params:
        hw, asrc_t, adst_t, mx_t = _pre(h, W, a_s.reshape(1, D),
                                        a_d.reshape(1, D))
        acc, den = _edge(hw, srcp, dstp, asrc_t.reshape(NP),
                         adst_t.reshape(NP), mx_t.reshape(D)[:16], z2, z1)
        h, pool_l = _post(acc[0], acc[1],
                          den[0].reshape(NP, 1), den[1].reshape(NP, 1),
                          b.reshape(1, D), g.reshape(1, D), be.reshape(1, D),
                          batch2)
        pooled.append(pool_l)
    return jnp.concatenate(pooled, axis=1), h[:N]


# speedup vs baseline: 30.3654x; 1.0352x over previous
import functools

import jax
import jax.numpy as jnp
from jax import lax
from jax.experimental import pallas as pl
from jax.experimental.pallas import tpu as pltpu
from jax.experimental.pallas import tpu_sc as plsc

N = 10000
E = 320000
D = 128
NG = 64
NP = 10240
CHUNK = 64
NTILES = 32
NCHUNK = 162
EP = NTILES * NCHUNK * CHUNK
STRIPE = NP // 16
PAD_NEG = -1e9
PAD_POS = 1e9


def _pre_body(h_ref, w_ref, asv_ref, adv_ref, hw_out, as_out, ad_out, mx_out):
    hw = jnp.dot(h_ref[...], w_ref[...], preferred_element_type=jnp.float32)
    hw_out[...] = hw
    a_s = jnp.sum(hw * asv_ref[...], axis=1, keepdims=True)
    a_d = jnp.sum(hw * adv_ref[...], axis=1, keepdims=True)
    valid = lax.broadcasted_iota(jnp.int32, (NP, 1), 0) < N
    as_out[...] = jnp.where(valid, a_s, PAD_NEG)
    ad_out[...] = jnp.where(valid, a_d, PAD_NEG)
    max_as = jnp.max(jnp.where(valid, a_s, PAD_NEG))
    mx_out[...] = jnp.zeros((1, D), jnp.float32) + max_as


_pre = pl.pallas_call(
    _pre_body,
    out_shape=(
        jax.ShapeDtypeStruct((NP, D), jnp.float32),
        jax.ShapeDtypeStruct((NP, 1), jnp.float32),
        jax.ShapeDtypeStruct((NP, 1), jnp.float32),
        jax.ShapeDtypeStruct((1, D), jnp.float32),
    ),
)


NBUF = 3


def _edge_body(h_hbm, src_hbm, dst_hbm, asrc_hbm, adst_hbm, mx_hbm,
               z2_hbm, z1_hbm, out_hbm, den_hbm,
               asrc_v, adst_v, mx_v, sidx_v, didx_v, rows_v, ex_v,
               acc_sh, den_sh, gsem, ssem, esem):
    cid = lax.axis_index("c")
    sid = lax.axis_index("s")
    wid = cid * 16 + sid
    rbase = sid * STRIPE
    ebase = wid * NCHUNK * CHUNK

    pltpu.sync_copy(asrc_hbm, asrc_v)
    pltpu.sync_copy(adst_hbm, adst_v)
    pltpu.sync_copy(mx_hbm, mx_v)
    pltpu.sync_copy(z2_hbm, acc_sh.at[pl.ds(rbase, STRIPE)])
    pltpu.sync_copy(z1_hbm, den_sh.at[pl.ds(rbase, STRIPE)])
    plsc.subcore_barrier()

    def load_idx(c, b):
        pltpu.sync_copy(src_hbm.at[pl.ds(ebase + c * CHUNK, CHUNK)],
                        sidx_v.at[b])
        pltpu.sync_copy(dst_hbm.at[pl.ds(ebase + c * CHUNK, CHUNK)],
                        didx_v.at[b])

    def gather(b):
        pltpu.async_copy(h_hbm.at[sidx_v.at[b]], rows_v.at[b], gsem.at[b])

    def wait_scatter(b):
        pltpu.make_async_copy(rows_v.at[b], acc_sh.at[didx_v.at[b]],
                              ssem.at[b]).wait()
        pltpu.make_async_copy(ex_v.at[b], den_sh.at[didx_v.at[b]],
                              esem.at[b]).wait()

    load_idx(0, 0)
    gather(0)
    mx = mx_v[pl.ds(0, 16)]

    def outer_body(o, carry):
        for b in range(NBUF):
            c = o * NBUF + b
            bn = (b + 1) % NBUF

            for g in range(CHUNK // 16):
                sv = sidx_v[b, pl.ds(g * 16, 16)]
                dv = didx_v[b, pl.ds(g * 16, 16)]
                asv = plsc.load_gather(asrc_v, [sv])
                adv = plsc.load_gather(adst_v, [dv])
                tb = mx + adv
                mv = jnp.maximum(tb, 0.2 * tb)
                t = asv + adv
                ex_v[b, pl.ds(g * 16, 16)] = (
                    jnp.exp(jnp.maximum(t, 0.2 * t) - mv))

            @pl.when(jnp.logical_and(c + 1 < NCHUNK, c >= NBUF - 1))
            def _():
                wait_scatter(bn)

            @pl.when(c + 1 < NCHUNK)
            def _():
                load_idx(c + 1, bn)
                gather(bn)

            pltpu.make_async_copy(h_hbm.at[sidx_v.at[b]], rows_v.at[b],
                                  gsem.at[b]).wait()

            def scale_body(e8, carry2):
                for u in range(8):
                    e = e8 * 8 + u
                    bex = plsc.load_gather(
                        ex_v.at[b], [jnp.full((16,), 0, jnp.int32) + e])
                    for j in range(D // 16):
                        rows_v[b, e, pl.ds(j * 16, 16)] = (
                            rows_v[b, e, pl.ds(j * 16, 16)] * bex)
                return carry2

            lax.fori_loop(0, CHUNK // 8, scale_body, 0)
            pltpu.async_copy(rows_v.at[b], acc_sh.at[didx_v.at[b]],
                             ssem.at[b], add=True)
            pltpu.async_copy(ex_v.at[b], den_sh.at[didx_v.at[b]],
                             esem.at[b], add=True)
        return carry

    lax.fori_loop(0, NCHUNK // NBUF, outer_body, 0)
    for k in range(NBUF):
        wait_scatter((NCHUNK - NBUF + k) % NBUF)
    plsc.subcore_barrier()
    pltpu.sync_copy(acc_sh.at[pl.ds(rbase, STRIPE)],
                    out_hbm.at[cid, pl.ds(rbase, STRIPE)])
    pltpu.sync_copy(den_sh.at[pl.ds(rbase, STRIPE)],
                    den_hbm.at[cid, pl.ds(rbase, STRIPE)])


_edge = functools.partial(
    pl.kernel,
    out_type=(
        jax.ShapeDtypeStruct((2, NP, D), jnp.float32),
        jax.ShapeDtypeStruct((2, NP), jnp.float32),
    ),
    mesh=plsc.VectorSubcoreMesh(core_axis_name="c", subcore_axis_name="s"),
    compiler_params=pltpu.CompilerParams(needs_layout_passes=False),
    scratch_types=[
        pltpu.VMEM((NP,), jnp.float32),
        pltpu.VMEM((NP,), jnp.float32),
        pltpu.VMEM((16,), jnp.float32),
        pltpu.VMEM((NBUF, CHUNK), jnp.int32),
        pltpu.VMEM((NBUF, CHUNK), jnp.int32),
        pltpu.VMEM((NBUF, CHUNK, D), jnp.float32),
        pltpu.VMEM((NBUF, CHUNK), jnp.float32),
        pltpu.VMEM_SHARED((NP, D), jnp.float32),
        pltpu.VMEM_SHARED((NP,), jnp.float32),
        pltpu.SemaphoreType.DMA((NBUF,)),
        pltpu.SemaphoreType.DMA((NBUF,)),
        pltpu.SemaphoreType.DMA((NBUF,)),
    ],
)(_edge_body)


def _post_body(a0_ref, a1_ref, d0_ref, d1_ref, bias_ref, gamma_ref,
               beta_ref, batch_ref, h_out, pool_out):
    acc = a0_ref[...] + a1_ref[...]
    den = d0_ref[...] + d1_ref[...]
    y = acc / (den + 1e-16) + bias_ref[...]
    y = jnp.where(y > 0, y, jnp.exp(jnp.minimum(y, 0.0)) - 1.0)
    valid = lax.broadcasted_iota(jnp.int32, (NP, D), 0) < N
    y = jnp.where(valid, y, 0.0)
    mu = jnp.sum(y, axis=0, keepdims=True) / N
    var = jnp.sum(y * y, axis=0, keepdims=True) / N - mu * mu
    hn = gamma_ref[...] * (y - mu) * lax.rsqrt(var + 1e-5) + beta_ref[...]
    hn = jnp.where(valid, hn, 0.0)
    h_out[...] = hn
    onehot = (lax.broadcasted_iota(jnp.int32, (NG, NP), 0)
              == batch_ref[...]).astype(jnp.float32)
    pool_out[...] = jnp.dot(onehot, hn, preferred_element_type=jnp.float32)


_post = pl.pallas_call(
    _post_body,
    out_shape=(
        jax.ShapeDtypeStruct((NP, D), jnp.float32),
        jax.ShapeDtypeStruct((NG, D), jnp.float32),
    ),
)


def kernel(x, edge_index, batch, W1, att_src1, att_dst1, bias1, gamma1, beta1,
           W2, att_src2, att_dst2, bias2, gamma2, beta2,
           W3, att_src3, att_dst3, bias3, gamma3, beta3):
    params = [(W1, att_src1, att_dst1, bias1, gamma1, beta1),
              (W2, att_src2, att_dst2, bias2, gamma2, beta2),
              (W3, att_src3, att_dst3, bias3, gamma3, beta3)]
    loop = jnp.arange(N, dtype=jnp.int32)
    srcp = jnp.concatenate(
        [edge_index[0], loop, jnp.full((EP - E - N,), N, jnp.int32)])
    dstp = jnp.concatenate(
        [edge_index[1], loop, jnp.full((EP - E - N,), N, jnp.int32)])
    batch2 = jnp.pad(batch, (0, NP - N), constant_values=NG).reshape(1, NP)
    z2 = jnp.zeros((STRIPE, D), jnp.float32)
    z1 = jnp.zeros((STRIPE,), jnp.float32)

    h = jnp.pad(x, ((0, NP - N), (0, 0)))
    pooled = []
    for (W, a_s, a_d, b, g, be) in params:
        hw, asrc_t, adst_t, mx_t = _pre(h, W, a_s.reshape(1, D),
                                        a_d.reshape(1, D))
        acc, den = _edge(hw, srcp, dstp, asrc_t.reshape(NP),
                         adst_t.reshape(NP), mx_t.reshape(D)[:16], z2, z1)
        h, pool_l = _post(acc[0], acc[1],
                          den[0].reshape(NP, 1), den[1].reshape(NP, 1),
                          b.reshape(1, D), g.reshape(1, D), be.reshape(1, D),
                          batch2)
        pooled.append(pool_l)
    return jnp.concatenate(pooled, axis=1), h[:N]


# speedup vs baseline: 38.5732x; 1.2703x over previous
import functools

import jax
import jax.numpy as jnp
from jax import lax
from jax.experimental import pallas as pl
from jax.experimental.pallas import tpu as pltpu
from jax.experimental.pallas import tpu_sc as plsc

N = 10000
E = 320000
D = 128
NG = 64
NP = 10240
CHUNK = 64
NTILES = 32
NCHUNK = 162
EP = NTILES * NCHUNK * CHUNK
STRIPE = NP // 16
PAD_NEG = -1e9
PAD_POS = 1e9


def _pre_body(h_ref, w_ref, asv_ref, adv_ref, hw_out, as_out, ad_out, mx_out):
    hw = jnp.dot(h_ref[...], w_ref[...], preferred_element_type=jnp.float32)
    hw_out[...] = hw
    a_s = jnp.sum(hw * asv_ref[...], axis=1, keepdims=True)
    a_d = jnp.sum(hw * adv_ref[...], axis=1, keepdims=True)
    valid = lax.broadcasted_iota(jnp.int32, (NP, 1), 0) < N
    as_out[...] = jnp.where(valid, a_s, PAD_NEG)
    ad_out[...] = jnp.where(valid, a_d, PAD_NEG)
    max_as = jnp.max(jnp.where(valid, a_s, PAD_NEG))
    mx_out[...] = jnp.zeros((1, D), jnp.float32) + max_as


_pre = pl.pallas_call(
    _pre_body,
    out_shape=(
        jax.ShapeDtypeStruct((NP, D), jnp.float32),
        jax.ShapeDtypeStruct((NP, 1), jnp.float32),
        jax.ShapeDtypeStruct((NP, 1), jnp.float32),
        jax.ShapeDtypeStruct((1, D), jnp.float32),
    ),
)


NBUF = 3
NIDX = 6


def _edge_body(h_hbm, src_hbm, dst_hbm, asrc_hbm, adst_hbm, mx_hbm,
               z2_hbm, z1_hbm, out_hbm, den_hbm,
               asrc_v, adst_v, mx_v, sidx_v, didx_v, rows_v, ex_v,
               acc_sh, den_sh, gsem, ssem, esem, isem):
    cid = lax.axis_index("c")
    sid = lax.axis_index("s")
    wid = cid * 16 + sid
    rbase = sid * STRIPE
    ebase = wid * NCHUNK * CHUNK

    pltpu.sync_copy(asrc_hbm, asrc_v)
    pltpu.sync_copy(adst_hbm, adst_v)
    pltpu.sync_copy(mx_hbm, mx_v)
    pltpu.sync_copy(z2_hbm, acc_sh.at[pl.ds(rbase, STRIPE)])
    pltpu.sync_copy(z1_hbm, den_sh.at[pl.ds(rbase, STRIPE)])
    plsc.subcore_barrier()

    def load_idx(c, bi):
        pltpu.async_copy(src_hbm.at[pl.ds(ebase + c * CHUNK, CHUNK)],
                         sidx_v.at[bi], isem.at[bi])
        pltpu.async_copy(dst_hbm.at[pl.ds(ebase + c * CHUNK, CHUNK)],
                         didx_v.at[bi], isem.at[bi])

    def wait_idx(c, bi):
        pltpu.make_async_copy(src_hbm.at[pl.ds(ebase + c * CHUNK, CHUNK)],
                              sidx_v.at[bi], isem.at[bi]).wait()
        pltpu.make_async_copy(dst_hbm.at[pl.ds(ebase + c * CHUNK, CHUNK)],
                              didx_v.at[bi], isem.at[bi]).wait()

    def gather(bi, b):
        pltpu.async_copy(h_hbm.at[sidx_v.at[bi]], rows_v.at[b], gsem.at[b])

    def wait_scatter(bi, b):
        pltpu.make_async_copy(rows_v.at[b], acc_sh.at[didx_v.at[bi]],
                              ssem.at[b]).wait()
        pltpu.make_async_copy(ex_v.at[b], den_sh.at[didx_v.at[bi]],
                              esem.at[b]).wait()

    for c0 in range(NIDX // 2):
        load_idx(c0, c0)
    wait_idx(0, 0)
    gather(0, 0)
    mx = mx_v[pl.ds(0, 16)]

    def outer_body(o, carry):
        for u in range(NIDX):
            c = o * NIDX + u
            bi = u
            b = u % NBUF
            bn = (u + 1) % NBUF
            bin_ = (u + 1) % NIDX

            for g in range(CHUNK // 16):
                sv = sidx_v[bi, pl.ds(g * 16, 16)]
                dv = didx_v[bi, pl.ds(g * 16, 16)]
                asv = plsc.load_gather(asrc_v, [sv])
                adv = plsc.load_gather(adst_v, [dv])
                tb = mx + adv
                mv = jnp.maximum(tb, 0.2 * tb)
                t = asv + adv
                ex_v[b, pl.ds(g * 16, 16)] = (
                    jnp.exp(jnp.maximum(t, 0.2 * t) - mv))

            @pl.when(jnp.logical_and(c + 1 < NCHUNK, c >= NBUF - 1))
            def _():
                wait_scatter((u + 1 - NBUF) % NIDX, bn)

            @pl.when(c + 1 < NCHUNK)
            def _():
                wait_idx(c + 1, bin_)
                gather(bin_, bn)

            @pl.when(c + NIDX // 2 < NCHUNK)
            def _():
                load_idx(c + NIDX // 2, (u + NIDX // 2) % NIDX)

            pltpu.make_async_copy(h_hbm.at[sidx_v.at[bi]], rows_v.at[b],
                                  gsem.at[b]).wait()

            def scale_body(e8, carry2):
                for uu in range(8):
                    e = e8 * 8 + uu
                    bex = plsc.load_gather(
                        ex_v.at[b], [jnp.full((16,), 0, jnp.int32) + e])
                    for j in range(D // 16):
                        rows_v[b, e, pl.ds(j * 16, 16)] = (
                            rows_v[b, e, pl.ds(j * 16, 16)] * bex)
                return carry2

            lax.fori_loop(0, CHUNK // 8, scale_body, 0)
            pltpu.async_copy(rows_v.at[b], acc_sh.at[didx_v.at[bi]],
                             ssem.at[b], add=True)
            pltpu.async_copy(ex_v.at[b], den_sh.at[didx_v.at[bi]],
                             esem.at[b], add=True)
        return carry

    lax.fori_loop(0, NCHUNK // NIDX, outer_body, 0)
    for k in range(NBUF):
        c = NCHUNK - NBUF + k
        wait_scatter(c % NIDX, c % NBUF)
    plsc.subcore_barrier()
    pltpu.sync_copy(acc_sh.at[pl.ds(rbase, STRIPE)],
                    out_hbm.at[cid, pl.ds(rbase, STRIPE)])
    pltpu.sync_copy(den_sh.at[pl.ds(rbase, STRIPE)],
                    den_hbm.at[cid, pl.ds(rbase, STRIPE)])


_edge = functools.partial(
    pl.kernel,
    out_type=(
        jax.ShapeDtypeStruct((2, NP, D), jnp.float32),
        jax.ShapeDtypeStruct((2, NP), jnp.float32),
    ),
    mesh=plsc.VectorSubcoreMesh(core_axis_name="c", subcore_axis_name="s"),
    compiler_params=pltpu.CompilerParams(needs_layout_passes=False),
    scratch_types=[
        pltpu.VMEM((NP,), jnp.float32),
        pltpu.VMEM((NP,), jnp.float32),
        pltpu.VMEM((16,), jnp.float32),
        pltpu.VMEM((NIDX, CHUNK), jnp.int32),
        pltpu.VMEM((NIDX, CHUNK), jnp.int32),
        pltpu.VMEM((NBUF, CHUNK, D), jnp.float32),
        pltpu.VMEM((NBUF, CHUNK), jnp.float32),
        pltpu.VMEM_SHARED((NP, D), jnp.float32),
        pltpu.VMEM_SHARED((NP,), jnp.float32),
        pltpu.SemaphoreType.DMA((NBUF,)),
        pltpu.SemaphoreType.DMA((NBUF,)),
        pltpu.SemaphoreType.DMA((NBUF,)),
        pltpu.SemaphoreType.DMA((NIDX,)),
    ],
)(_edge_body)


def _post_body(a0_ref, a1_ref, d0_ref, d1_ref, bias_ref, gamma_ref,
               beta_ref, batch_ref, h_out, pool_out):
    acc = a0_ref[...] + a1_ref[...]
    den = d0_ref[...] + d1_ref[...]
    y = acc / (den + 1e-16) + bias_ref[...]
    y = jnp.where(y > 0, y, jnp.exp(jnp.minimum(y, 0.0)) - 1.0)
    valid = lax.broadcasted_iota(jnp.int32, (NP, D), 0) < N
    y = jnp.where(valid, y, 0.0)
    mu = jnp.sum(y, axis=0, keepdims=True) / N
    var = jnp.sum(y * y, axis=0, keepdims=True) / N - mu * mu
    hn = gamma_ref[...] * (y - mu) * lax.rsqrt(var + 1e-5) + beta_ref[...]
    hn = jnp.where(valid, hn, 0.0)
    h_out[...] = hn
    onehot = (lax.broadcasted_iota(jnp.int32, (NG, NP), 0)
              == batch_ref[...]).astype(jnp.float32)
    pool_out[...] = jnp.dot(onehot, hn, preferred_element_type=jnp.float32)


_post = pl.pallas_call(
    _post_body,
    out_shape=(
        jax.ShapeDtypeStruct((NP, D), jnp.float32),
        jax.ShapeDtypeStruct((NG, D), jnp.float32),
    ),
)


def kernel(x, edge_index, batch, W1, att_src1, att_dst1, bias1, gamma1, beta1,
           W2, att_src2, att_dst2, bias2, gamma2, beta2,
           W3, att_src3, att_dst3, bias3, gamma3, beta3):
    params = [(W1, att_src1, att_dst1, bias1, gamma1, beta1),
              (W2, att_src2, att_dst2, bias2, gamma2, beta2),
              (W3, att_src3, att_dst3, bias3, gamma3, beta3)]
    loop = jnp.arange(N, dtype=jnp.int32)
    srcp = jnp.concatenate(
        [edge_index[0], loop, jnp.full((EP - E - N,), N, jnp.int32)])
    dstp = jnp.concatenate(
        [edge_index[1], loop, jnp.full((EP - E - N,), N, jnp.int32)])
    batch2 = jnp.pad(batch, (0, NP - N), constant_values=NG).reshape(1, NP)
    z2 = jnp.zeros((STRIPE, D), jnp.float32)
    z1 = jnp.zeros((STRIPE,), jnp.float32)

    h = jnp.pad(x, ((0, NP - N), (0, 0)))
    pooled = []
    for (W, a_s, a_d, b, g, be) in params:
        hw, asrc_t, adst_t, mx_t = _pre(h, W, a_s.reshape(1, D),
                                        a_d.reshape(1, D))
        acc, den = _edge(hw, srcp, dstp, asrc_t.reshape(NP),
                         adst_t.reshape(NP), mx_t.reshape(D)[:16], z2, z1)
        h, pool_l = _post(acc[0], acc[1],
                          den[0].reshape(NP, 1), den[1].reshape(NP, 1),
                          b.reshape(1, D), g.reshape(1, D), be.reshape(1, D),
                          batch2)
        pooled.append(pool_l)
    return jnp.concatenate(pooled, axis=1), h[:N]
